# baseline (speedup 1.0000x reference)
import functools

import jax
import jax.numpy as jnp
from jax import lax
from jax.experimental import pallas as pl
from jax.experimental.pallas import tpu as pltpu
from jax.experimental.pallas import tpu_sc as plsc

_VOCAB = 100000
_HIDDEN = 128
_MAX_LEN = 2048
_NC = 2
_NS = 16
_NW = _NC * _NS
_LANES = 16


def _emb_kernel(ids_hbm, tt_hbm, word_hbm, pos_hbm, type_hbm, out_hbm,
                idx_v, tti_v, we_v, pe_v, te_v, sem):
    wid = lax.axis_index("s") * _NC + lax.axis_index("c")
    tok_per_w = 256
    base = wid * tok_per_w
    s0 = lax.rem(base, _MAX_LEN)

    pltpu.sync_copy(ids_hbm.at[wid], idx_v)
    pltpu.sync_copy(tt_hbm.at[wid], tti_v)

    g0 = pltpu.async_copy(word_hbm.at[idx_v.at[0]], we_v.at[pl.ds(0, 128)], sem)
    g1 = pltpu.async_copy(word_hbm.at[idx_v.at[1]], we_v.at[pl.ds(128, 128)], sem)
    g2 = pltpu.async_copy(type_hbm.at[tti_v.at[0]], te_v.at[pl.ds(0, 128)], sem)
    g3 = pltpu.async_copy(type_hbm.at[tti_v.at[1]], te_v.at[pl.ds(128, 128)], sem)
    pltpu.sync_copy(pos_hbm.at[pl.ds(s0, tok_per_w)], pe_v)
    g0.wait()
    g1.wait()
    g2.wait()
    g3.wait()

    def body(i, _):
        for j in range(_HIDDEN // _LANES):
            c = j * _LANES
            we_v[i, pl.ds(c, _LANES)] = (
                we_v[i, pl.ds(c, _LANES)]
                + pe_v[i, pl.ds(c, _LANES)]
                + te_v[i, pl.ds(c, _LANES)]
            )
        return _

    lax.fori_loop(0, tok_per_w, body, None)

    pltpu.sync_copy(we_v, out_hbm.at[pl.ds(base, tok_per_w)])


@jax.jit
def _embedding_sum(ids3, tt3, word_emb, pos_emb, type_emb):
    mesh = plsc.VectorSubcoreMesh(core_axis_name="c", subcore_axis_name="s")
    kfn = functools.partial(
        pl.kernel,
        mesh=mesh,
        out_type=jax.ShapeDtypeStruct((_NW * 256, _HIDDEN), jnp.float32),
        scratch_types=[
            pltpu.VMEM((2, 128), jnp.int32),
            pltpu.VMEM((2, 128), jnp.int32),
            pltpu.VMEM((256, _HIDDEN), jnp.float32),
            pltpu.VMEM((256, _HIDDEN), jnp.float32),
            pltpu.VMEM((256, _HIDDEN), jnp.float32),
            pltpu.SemaphoreType.DMA,
        ],
    )(_emb_kernel)
    return kfn(ids3, tt3, word_emb, pos_emb, type_emb)


def kernel(input_ids, token_type_ids, word_emb, pos_emb, type_emb):
    b, s = input_ids.shape
    ids3 = input_ids.astype(jnp.int32).reshape(_NW, 2, 128)
    tt3 = token_type_ids.astype(jnp.int32).reshape(_NW, 2, 128)
    out = _embedding_sum(ids3, tt3, word_emb, pos_emb, type_emb)
    return out.reshape(b, s, _HIDDEN)


# speedup vs baseline: 4.9726x; 4.9726x over previous
import functools

import jax
import jax.numpy as jnp
from jax import lax
from jax.experimental import pallas as pl
from jax.experimental.pallas import tpu as pltpu
from jax.experimental.pallas import tpu_sc as plsc

_VOCAB = 100000
_HIDDEN = 128
_MAX_LEN = 2048
_NC = 2
_NS = 16
_NW = _NC * _NS
_LANES = 16
_TOK_PER_W = 256


def _emb_kernel(ids_hbm, tt_hbm, word_hbm, pos_hbm, type_hbm, out_hbm,
                idx_v, tti_v, we_v, pe_v, ty_v, sem):
    wid = lax.axis_index("s") * _NC + lax.axis_index("c")
    base = wid * _TOK_PER_W
    s0 = lax.rem(base, _MAX_LEN)

    pltpu.sync_copy(ids_hbm.at[wid], idx_v)
    pltpu.sync_copy(tt_hbm.at[wid], tti_v)

    g0 = pltpu.async_copy(word_hbm.at[idx_v.at[0]], we_v.at[pl.ds(0, 128)], sem)
    g1 = pltpu.async_copy(word_hbm.at[idx_v.at[1]], we_v.at[pl.ds(128, 128)], sem)
    pltpu.sync_copy(type_hbm, ty_v)
    pltpu.sync_copy(pos_hbm.at[pl.ds(s0, _TOK_PER_W)], pe_v)
    g0.wait()
    g1.wait()

    t0 = [ty_v[0, pl.ds(j * _LANES, _LANES)] for j in range(_HIDDEN // _LANES)]
    td = [ty_v[1, pl.ds(j * _LANES, _LANES)] - t0[j]
          for j in range(_HIDDEN // _LANES)]

    def body(g, _):
        gbase = g * _LANES
        ttf = tti_v[pl.ds(gbase, _LANES)].astype(jnp.float32)
        for k in range(_LANES):
            i = gbase + k
            ttv = jnp.full((_LANES,), ttf[k], jnp.float32)
            for j in range(_HIDDEN // _LANES):
                c = j * _LANES
                we_v[i, pl.ds(c, _LANES)] = (
                    we_v[i, pl.ds(c, _LANES)] + pe_v[i, pl.ds(c, _LANES)]
                    + (t0[j] + ttv * td[j])
                )
        return _

    lax.fori_loop(0, _TOK_PER_W // _LANES, body, None)

    pltpu.sync_copy(we_v, out_hbm.at[pl.ds(base, _TOK_PER_W)])


@jax.jit
def _embedding_sum(ids3, tt2, word_emb, pos_emb, type_emb):
    mesh = plsc.VectorSubcoreMesh(core_axis_name="c", subcore_axis_name="s")
    kfn = functools.partial(
        pl.kernel,
        mesh=mesh,
        out_type=jax.ShapeDtypeStruct((_NW * _TOK_PER_W, _HIDDEN), jnp.float32),
        scratch_types=[
            pltpu.VMEM((2, 128), jnp.int32),
            pltpu.VMEM((_TOK_PER_W,), jnp.int32),
            pltpu.VMEM((_TOK_PER_W, _HIDDEN), jnp.float32),
            pltpu.VMEM((_TOK_PER_W, _HIDDEN), jnp.float32),
            pltpu.VMEM((2, _HIDDEN), jnp.float32),
            pltpu.SemaphoreType.DMA,
        ],
    )(_emb_kernel)
    return kfn(ids3, tt2, word_emb, pos_emb, type_emb)


def kernel(input_ids, token_type_ids, word_emb, pos_emb, type_emb):
    b, s = input_ids.shape
    ids3 = input_ids.astype(jnp.int32).reshape(_NW, 2, 128)
    tt2 = token_type_ids.astype(jnp.int32).reshape(_NW, _TOK_PER_W)
    out = _embedding_sum(ids3, tt2, word_emb, pos_emb, type_emb)
    return out.reshape(b, s, _HIDDEN)
